# baseline (speedup 1.0000x reference)
import functools

import jax
import jax.numpy as jnp
from jax import lax
from jax.experimental import pallas as pl
from jax.experimental.pallas import tpu as pltpu
from jax.experimental.pallas import tpu_sc as plsc

VOCAB = 100001
EMBED = 64
BATCH = 1024
CTX = 20

_INFO = plsc.get_sparse_core_info()
_NC = _INFO.num_cores
_NS = _INFO.num_subcores
_NW = _NC * _NS
_BPW = BATCH // _NW
_IPW = _BPW * CTX
_LANES = 16
_DCH = EMBED // _LANES


_ROW = 128


def _pool_body(ctx_hbm, table_hbm, out_hbm, idx_v, rows_v, acc_v, sem):
    wid = lax.axis_index("s") * _NC + lax.axis_index("c")
    base = wid * _IPW
    pltpu.sync_copy(ctx_hbm.at[pl.ds(base, _IPW)], idx_v)
    pltpu.async_copy(table_hbm.at[idx_v], rows_v, sem).wait()

    iota = lax.iota(jnp.int32, _LANES)

    def body(r, _):
        row0 = r * CTX
        accs = [jnp.zeros((_LANES,), jnp.float32) for _ in range(4)]
        for j in range(CTX):
            for c in range(2):
                packed = rows_v[row0 + j, pl.ds(c * 32, 32)]
                lo, hi = plsc.unpack(packed, format=plsc.PackFormat.INTERLEAVED)
                accs[2 * c] = accs[2 * c] + lo
                accs[2 * c + 1] = accs[2 * c + 1] + hi
        rbase = r * EMBED
        for c in range(2):
            plsc.store_scatter(
                acc_v, [rbase + c * 32 + 2 * iota], accs[2 * c] * (1.0 / CTX)
            )
            plsc.store_scatter(
                acc_v, [rbase + c * 32 + 2 * iota + 1], accs[2 * c + 1] * (1.0 / CTX)
            )
        return 0

    lax.fori_loop(0, _BPW, body, 0)
    pltpu.sync_copy(acc_v, out_hbm.at[pl.ds(wid * _BPW * EMBED, _BPW * EMBED)])


_pool = functools.partial(
    pl.kernel,
    out_type=jax.ShapeDtypeStruct((BATCH * EMBED,), jnp.float32),
    mesh=plsc.VectorSubcoreMesh(core_axis_name="c", subcore_axis_name="s"),
    scratch_types=[
        pltpu.VMEM((_IPW,), jnp.int32),
        pltpu.VMEM((_IPW, _ROW), jnp.bfloat16),
        pltpu.VMEM((_BPW * EMBED,), jnp.float32),
        pltpu.SemaphoreType.DMA,
    ],
    compiler_params=pltpu.CompilerParams(
        use_tc_tiling_on_sc=False, needs_layout_passes=False
    ),
)(_pool_body)


_TP_VT = 4096
_TP_STEPS = pl.cdiv(VOCAB, _TP_VT)


def _tpad_body(tt_ref, out_ref):
    t = jnp.transpose(tt_ref[...], (1, 0)).astype(jnp.bfloat16)
    out_ref[...] = jnp.concatenate(
        [t, jnp.zeros((t.shape[0], _ROW - EMBED), jnp.bfloat16)], axis=1
    )


def _tpad(tt):
    return pl.pallas_call(
        _tpad_body,
        grid=(_TP_STEPS,),
        in_specs=[pl.BlockSpec((EMBED, _TP_VT), lambda i: (0, i))],
        out_specs=pl.BlockSpec((_TP_VT, _ROW), lambda i: (i, 0)),
        out_shape=jax.ShapeDtypeStruct((VOCAB, _ROW), jnp.bfloat16),
    )(tt)


_VT = 4096
_NSTEPS = pl.cdiv(VOCAB, _VT)


def _proj_body(avg_ref, wt_ref, b_ref, out_ref):
    bt = jnp.transpose(b_ref[...], (1, 0))
    out_ref[...] = lax.dot_general(
        wt_ref[...], avg_ref[...],
        (((0,), (1,)), ((), ())),
        preferred_element_type=jnp.float32,
    ) + bt


def _projection(avg, Wt, b2):
    return pl.pallas_call(
        _proj_body,
        grid=(_NSTEPS,),
        in_specs=[
            pl.BlockSpec((BATCH, EMBED), lambda i: (0, 0)),
            pl.BlockSpec((EMBED, _VT), lambda i: (0, i)),
            pl.BlockSpec((1, _VT), lambda i: (0, i)),
        ],
        out_specs=pl.BlockSpec((_VT, BATCH), lambda i: (i, 0)),
        out_shape=jax.ShapeDtypeStruct((VOCAB, BATCH), jnp.float32),
    )(avg, Wt, b2)


def kernel(context, table, W, b):
    ctx_flat = context.reshape(-1)
    table_p = _tpad(table.T)
    avg_flat = _pool(ctx_flat, table_p)
    avg = avg_flat.reshape(BATCH, EMBED)
    yt = _projection(avg, W.T, b.reshape(1, VOCAB))
    return yt.T


# speedup vs baseline: 1.5207x; 1.5207x over previous
import functools

import jax
import jax.numpy as jnp
from jax import lax
from jax.experimental import pallas as pl
from jax.experimental.pallas import tpu as pltpu
from jax.experimental.pallas import tpu_sc as plsc

VOCAB = 100001
EMBED = 64
BATCH = 1024
CTX = 20

_INFO = plsc.get_sparse_core_info()
_NC = _INFO.num_cores
_NS = _INFO.num_subcores
_NW = _NC * _NS
_BPW = BATCH // _NW
_IPW = _BPW * CTX
_LANES = 16
_DCH = EMBED // _LANES


_ROW = 128


def _pool_body(ctx_hbm, table_hbm, out_hbm, idx_v, rows_v, acc_v, sem):
    wid = lax.axis_index("s") * _NC + lax.axis_index("c")
    base = wid * _IPW
    pltpu.sync_copy(ctx_hbm.at[pl.ds(base, _IPW)], idx_v)
    pltpu.async_copy(table_hbm.at[idx_v], rows_v, sem).wait()

    def body(r, _):
        row0 = r * CTX
        for c in range(_DCH):
            acc = rows_v[row0, pl.ds(c * _LANES, _LANES)]
            for j in range(1, CTX):
                acc = acc + rows_v[row0 + j, pl.ds(c * _LANES, _LANES)]
            acc_v[pl.ds(r * EMBED + c * _LANES, _LANES)] = acc * (1.0 / CTX)
        return 0

    lax.fori_loop(0, _BPW, body, 0)
    pltpu.sync_copy(acc_v, out_hbm.at[pl.ds(wid * _BPW * EMBED, _BPW * EMBED)])


_pool = functools.partial(
    pl.kernel,
    out_type=jax.ShapeDtypeStruct((BATCH * EMBED,), jnp.float32),
    mesh=plsc.VectorSubcoreMesh(core_axis_name="c", subcore_axis_name="s"),
    scratch_types=[
        pltpu.VMEM((_IPW,), jnp.int32),
        pltpu.VMEM((_IPW, _ROW), jnp.float32),
        pltpu.VMEM((_BPW * EMBED,), jnp.float32),
        pltpu.SemaphoreType.DMA,
    ],
    compiler_params=pltpu.CompilerParams(
        use_tc_tiling_on_sc=False, needs_layout_passes=False
    ),
)(_pool_body)


_TP_VT = 4096
_TP_STEPS = pl.cdiv(VOCAB, _TP_VT)


def _tpad_body(tt_ref, out_ref):
    t = jnp.transpose(tt_ref[...], (1, 0))
    out_ref[...] = jnp.concatenate(
        [t, jnp.zeros((t.shape[0], _ROW - EMBED), jnp.float32)], axis=1
    )


def _tpad(tt):
    return pl.pallas_call(
        _tpad_body,
        grid=(_TP_STEPS,),
        in_specs=[pl.BlockSpec((EMBED, _TP_VT), lambda i: (0, i))],
        out_specs=pl.BlockSpec((_TP_VT, _ROW), lambda i: (i, 0)),
        out_shape=jax.ShapeDtypeStruct((VOCAB, _ROW), jnp.float32),
    )(tt)


_VT = 4096
_NSTEPS = pl.cdiv(VOCAB, _VT)


def _proj_body(avg_ref, wt_ref, b_ref, out_ref):
    bt = jnp.transpose(b_ref[...], (1, 0))
    out_ref[...] = lax.dot_general(
        wt_ref[...], avg_ref[...],
        (((0,), (1,)), ((), ())),
        preferred_element_type=jnp.float32,
    ) + bt


def _projection(avg, Wt, b2):
    return pl.pallas_call(
        _proj_body,
        grid=(_NSTEPS,),
        in_specs=[
            pl.BlockSpec((BATCH, EMBED), lambda i: (0, 0)),
            pl.BlockSpec((EMBED, _VT), lambda i: (0, i)),
            pl.BlockSpec((1, _VT), lambda i: (0, i)),
        ],
        out_specs=pl.BlockSpec((_VT, BATCH), lambda i: (i, 0)),
        out_shape=jax.ShapeDtypeStruct((VOCAB, BATCH), jnp.float32),
    )(avg, Wt, b2)


def kernel(context, table, W, b):
    ctx_flat = context.reshape(-1)
    table_p = _tpad(table.T)
    avg_flat = _pool(ctx_flat, table_p)
    avg = avg_flat.reshape(BATCH, EMBED)
    yt = _projection(avg, W.T, b.reshape(1, VOCAB))
    return yt.T


# speedup vs baseline: 1.5775x; 1.0373x over previous
import functools

import jax
import jax.numpy as jnp
from jax import lax
from jax.experimental import pallas as pl
from jax.experimental.pallas import tpu as pltpu
from jax.experimental.pallas import tpu_sc as plsc

VOCAB = 100001
EMBED = 64
BATCH = 1024
CTX = 20

_INFO = plsc.get_sparse_core_info()
_NC = _INFO.num_cores
_NS = _INFO.num_subcores
_NW = _NC * _NS
_BPW = BATCH // _NW
_IPW = _BPW * CTX
_LANES = 16
_DCH = EMBED // _LANES


_ROW = 128


def _pool_body(ctx_hbm, table_hbm, out_hbm, idx_v, rows_v, acc_v, sem):
    wid = lax.axis_index("s") * _NC + lax.axis_index("c")
    base = wid * _IPW
    pltpu.sync_copy(ctx_hbm.at[pl.ds(base, _IPW)], idx_v)
    pltpu.async_copy(table_hbm.at[idx_v], rows_v, sem).wait()

    def body(r, _):
        row0 = r * CTX
        for c in range(_DCH):
            acc = rows_v[row0, pl.ds(c * _LANES, _LANES)]
            for j in range(1, CTX):
                acc = acc + rows_v[row0 + j, pl.ds(c * _LANES, _LANES)]
            acc_v[pl.ds(r * EMBED + c * _LANES, _LANES)] = acc * (1.0 / CTX)
        return 0

    lax.fori_loop(0, _BPW, body, 0)
    pltpu.sync_copy(acc_v, out_hbm.at[pl.ds(wid * _BPW * EMBED, _BPW * EMBED)])


_pool = functools.partial(
    pl.kernel,
    out_type=jax.ShapeDtypeStruct((BATCH * EMBED,), jnp.float32),
    mesh=plsc.VectorSubcoreMesh(core_axis_name="c", subcore_axis_name="s"),
    scratch_types=[
        pltpu.VMEM((_IPW,), jnp.int32),
        pltpu.VMEM((_IPW, _ROW), jnp.float32),
        pltpu.VMEM((_BPW * EMBED,), jnp.float32),
        pltpu.SemaphoreType.DMA,
    ],
    compiler_params=pltpu.CompilerParams(
        use_tc_tiling_on_sc=False, needs_layout_passes=False
    ),
)(_pool_body)


_TP_VT = 8192
_TP_STEPS = pl.cdiv(VOCAB, _TP_VT)


def _tpad_body(tt_ref, out_ref):
    t = jnp.transpose(tt_ref[...], (1, 0))
    out_ref[...] = jnp.concatenate(
        [t, jnp.zeros((t.shape[0], _ROW - EMBED), jnp.float32)], axis=1
    )


def _tpad(tt):
    return pl.pallas_call(
        _tpad_body,
        grid=(_TP_STEPS,),
        in_specs=[pl.BlockSpec((EMBED, _TP_VT), lambda i: (0, i))],
        out_specs=pl.BlockSpec((_TP_VT, _ROW), lambda i: (i, 0)),
        out_shape=jax.ShapeDtypeStruct((VOCAB, _ROW), jnp.float32),
    )(tt)


_VT = 4096
_NSTEPS = pl.cdiv(VOCAB, _VT)


def _proj_body(avg_ref, wt_ref, b_ref, out_ref):
    bt = jnp.transpose(b_ref[...], (1, 0))
    out_ref[...] = lax.dot_general(
        wt_ref[...], avg_ref[...],
        (((0,), (1,)), ((), ())),
        preferred_element_type=jnp.float32,
    ) + bt


def _projection(avg, Wt, b2):
    return pl.pallas_call(
        _proj_body,
        grid=(_NSTEPS,),
        in_specs=[
            pl.BlockSpec((BATCH, EMBED), lambda i: (0, 0)),
            pl.BlockSpec((EMBED, _VT), lambda i: (0, i)),
            pl.BlockSpec((1, _VT), lambda i: (0, i)),
        ],
        out_specs=pl.BlockSpec((_VT, BATCH), lambda i: (i, 0)),
        out_shape=jax.ShapeDtypeStruct((VOCAB, BATCH), jnp.float32),
    )(avg, Wt, b2)


def kernel(context, table, W, b):
    ctx_flat = context.reshape(-1)
    table_p = _tpad(table.T)
    avg_flat = _pool(ctx_flat, table_p)
    avg = avg_flat.reshape(BATCH, EMBED)
    yt = _projection(avg, W.T, b.reshape(1, VOCAB))
    return yt.T


# speedup vs baseline: 1.5912x; 1.0087x over previous
import functools

import jax
import jax.numpy as jnp
from jax import lax
from jax.experimental import pallas as pl
from jax.experimental.pallas import tpu as pltpu
from jax.experimental.pallas import tpu_sc as plsc

VOCAB = 100001
EMBED = 64
BATCH = 1024
CTX = 20

_INFO = plsc.get_sparse_core_info()
_NC = _INFO.num_cores
_NS = _INFO.num_subcores
_NW = _NC * _NS
_BPW = BATCH // _NW
_IPW = _BPW * CTX
_LANES = 16
_DCH = EMBED // _LANES


_ROW = 128


def _pool_body(ctx_hbm, table_hbm, out_hbm, idx_v, rows_v, acc_v, sem):
    wid = lax.axis_index("s") * _NC + lax.axis_index("c")
    base = wid * _IPW
    pltpu.sync_copy(ctx_hbm.at[pl.ds(base, _IPW)], idx_v)
    pltpu.async_copy(table_hbm.at[idx_v], rows_v, sem).wait()

    def body(r, _):
        row0 = r * CTX
        for c in range(_DCH):
            acc = rows_v[row0, pl.ds(c * _LANES, _LANES)]
            for j in range(1, CTX):
                acc = acc + rows_v[row0 + j, pl.ds(c * _LANES, _LANES)]
            acc_v[pl.ds(r * EMBED + c * _LANES, _LANES)] = acc * (1.0 / CTX)
        return 0

    lax.fori_loop(0, _BPW, body, 0)
    pltpu.sync_copy(acc_v, out_hbm.at[pl.ds(wid * _BPW * EMBED, _BPW * EMBED)])


_pool = functools.partial(
    pl.kernel,
    out_type=jax.ShapeDtypeStruct((BATCH * EMBED,), jnp.float32),
    mesh=plsc.VectorSubcoreMesh(core_axis_name="c", subcore_axis_name="s"),
    scratch_types=[
        pltpu.VMEM((_IPW,), jnp.int32),
        pltpu.VMEM((_IPW, _ROW), jnp.float32),
        pltpu.VMEM((_BPW * EMBED,), jnp.float32),
        pltpu.SemaphoreType.DMA,
    ],
    compiler_params=pltpu.CompilerParams(
        use_tc_tiling_on_sc=False, needs_layout_passes=False
    ),
)(_pool_body)


_TP_VT = 16384
_TP_STEPS = pl.cdiv(VOCAB, _TP_VT)


def _tpad_body(tt_ref, out_ref):
    t = jnp.transpose(tt_ref[...], (1, 0))
    out_ref[...] = jnp.concatenate(
        [t, jnp.zeros((t.shape[0], _ROW - EMBED), jnp.float32)], axis=1
    )


def _tpad(tt):
    return pl.pallas_call(
        _tpad_body,
        grid=(_TP_STEPS,),
        in_specs=[pl.BlockSpec((EMBED, _TP_VT), lambda i: (0, i))],
        out_specs=pl.BlockSpec((_TP_VT, _ROW), lambda i: (i, 0)),
        out_shape=jax.ShapeDtypeStruct((VOCAB, _ROW), jnp.float32),
    )(tt)


_VT = 4096
_NSTEPS = pl.cdiv(VOCAB, _VT)


def _proj_body(avg_ref, wt_ref, b_ref, out_ref):
    bt = jnp.transpose(b_ref[...], (1, 0))
    out_ref[...] = lax.dot_general(
        wt_ref[...], avg_ref[...],
        (((0,), (1,)), ((), ())),
        preferred_element_type=jnp.float32,
    ) + bt


def _projection(avg, Wt, b2):
    return pl.pallas_call(
        _proj_body,
        grid=(_NSTEPS,),
        in_specs=[
            pl.BlockSpec((BATCH, EMBED), lambda i: (0, 0)),
            pl.BlockSpec((EMBED, _VT), lambda i: (0, i)),
            pl.BlockSpec((1, _VT), lambda i: (0, i)),
        ],
        out_specs=pl.BlockSpec((_VT, BATCH), lambda i: (i, 0)),
        out_shape=jax.ShapeDtypeStruct((VOCAB, BATCH), jnp.float32),
    )(avg, Wt, b2)


def kernel(context, table, W, b):
    ctx_flat = context.reshape(-1)
    table_p = _tpad(table.T)
    avg_flat = _pool(ctx_flat, table_p)
    avg = avg_flat.reshape(BATCH, EMBED)
    yt = _projection(avg, W.T, b.reshape(1, VOCAB))
    return yt.T


# speedup vs baseline: 1.5993x; 1.0051x over previous
import functools

import jax
import jax.numpy as jnp
from jax import lax
from jax.experimental import pallas as pl
from jax.experimental.pallas import tpu as pltpu
from jax.experimental.pallas import tpu_sc as plsc

VOCAB = 100001
EMBED = 64
BATCH = 1024
CTX = 20

_INFO = plsc.get_sparse_core_info()
_NC = _INFO.num_cores
_NS = _INFO.num_subcores
_NW = _NC * _NS
_BPW = BATCH // _NW
_IPW = _BPW * CTX
_LANES = 16
_DCH = EMBED // _LANES


_ROW = 128


def _pool_body(ctx_hbm, table_hbm, out_hbm, idx_v, rows_v, acc_v, sem):
    wid = lax.axis_index("s") * _NC + lax.axis_index("c")
    base = wid * _IPW
    pltpu.sync_copy(ctx_hbm.at[pl.ds(base, _IPW)], idx_v)
    pltpu.async_copy(table_hbm.at[idx_v], rows_v, sem).wait()

    def body(r, _):
        row0 = r * CTX
        for c in range(_DCH):
            acc = rows_v[row0, pl.ds(c * _LANES, _LANES)]
            for j in range(1, CTX):
                acc = acc + rows_v[row0 + j, pl.ds(c * _LANES, _LANES)]
            acc_v[pl.ds(r * EMBED + c * _LANES, _LANES)] = acc * (1.0 / CTX)
        return 0

    lax.fori_loop(0, _BPW, body, 0)
    pltpu.sync_copy(acc_v, out_hbm.at[pl.ds(wid * _BPW * EMBED, _BPW * EMBED)])


_pool = functools.partial(
    pl.kernel,
    out_type=jax.ShapeDtypeStruct((BATCH * EMBED,), jnp.float32),
    mesh=plsc.VectorSubcoreMesh(core_axis_name="c", subcore_axis_name="s"),
    scratch_types=[
        pltpu.VMEM((_IPW,), jnp.int32),
        pltpu.VMEM((_IPW, _ROW), jnp.float32),
        pltpu.VMEM((_BPW * EMBED,), jnp.float32),
        pltpu.SemaphoreType.DMA,
    ],
    compiler_params=pltpu.CompilerParams(
        use_tc_tiling_on_sc=False, needs_layout_passes=False
    ),
)(_pool_body)


_TP_VT = 32768
_TP_STEPS = pl.cdiv(VOCAB, _TP_VT)


def _tpad_body(tt_ref, out_ref):
    t = jnp.transpose(tt_ref[...], (1, 0))
    out_ref[...] = jnp.concatenate(
        [t, jnp.zeros((t.shape[0], _ROW - EMBED), jnp.float32)], axis=1
    )


def _tpad(tt):
    return pl.pallas_call(
        _tpad_body,
        grid=(_TP_STEPS,),
        in_specs=[pl.BlockSpec((EMBED, _TP_VT), lambda i: (0, i))],
        out_specs=pl.BlockSpec((_TP_VT, _ROW), lambda i: (i, 0)),
        out_shape=jax.ShapeDtypeStruct((VOCAB, _ROW), jnp.float32),
    )(tt)


_VT = 4096
_NSTEPS = pl.cdiv(VOCAB, _VT)


def _proj_body(avg_ref, wt_ref, b_ref, out_ref):
    bt = jnp.transpose(b_ref[...], (1, 0))
    out_ref[...] = lax.dot_general(
        wt_ref[...], avg_ref[...],
        (((0,), (1,)), ((), ())),
        preferred_element_type=jnp.float32,
    ) + bt


def _projection(avg, Wt, b2):
    return pl.pallas_call(
        _proj_body,
        grid=(_NSTEPS,),
        in_specs=[
            pl.BlockSpec((BATCH, EMBED), lambda i: (0, 0)),
            pl.BlockSpec((EMBED, _VT), lambda i: (0, i)),
            pl.BlockSpec((1, _VT), lambda i: (0, i)),
        ],
        out_specs=pl.BlockSpec((_VT, BATCH), lambda i: (i, 0)),
        out_shape=jax.ShapeDtypeStruct((VOCAB, BATCH), jnp.float32),
    )(avg, Wt, b2)


def kernel(context, table, W, b):
    ctx_flat = context.reshape(-1)
    table_p = _tpad(table.T)
    avg_flat = _pool(ctx_flat, table_p)
    avg = avg_flat.reshape(BATCH, EMBED)
    yt = _projection(avg, W.T, b.reshape(1, VOCAB))
    return yt.T
